# baseline (speedup 1.0000x reference)
import functools

import jax
import jax.numpy as jnp
from jax import lax
from jax.experimental import pallas as pl
from jax.experimental.pallas import tpu as pltpu
from jax.experimental.pallas import tpu_sc as plsc

PAIRS = 128
LANES = 16
PENALTY = 10.0
CHUNK = 4096
UNROLL = 8


def _body(batch, pairs_per_w, nc,
          comp_hbm, idx_i_hbm, idx_j_hbm, out_hbm,
          ii_v, jj_v, iv_list, rows_v, acc_v, sem0, sem1):
    wid = lax.axis_index("s") * nc + lax.axis_index("c")
    p0 = wid * pairs_per_w

    pltpu.sync_copy(idx_i_hbm, ii_v)
    pltpu.sync_copy(idx_j_hbm, jj_v)

    iota = lax.iota(jnp.int32, LANES)
    first2 = iota < 2
    for k in range(pairs_per_w):
        pv = jnp.full((LANES,), p0 + k, jnp.int32)
        gi = plsc.load_gather(ii_v, [pv])
        gj = plsc.load_gather(jj_v, [pv])
        ivec = jnp.where(iota == 0, gi, gj)
        iv_list[k][...] = ivec

    nchunks = batch // CHUNK
    nsteps = pairs_per_w * nchunks
    sems = [sem0, sem1]

    def start(step, slot):
        k, c = step // nchunks, step % nchunks
        return pltpu.async_copy(
            comp_hbm.at[iv_list[k].at[pl.ds(0, 2)], pl.ds(c * CHUNK, CHUNK)],
            rows_v.at[slot], sems[slot])

    def fma_chunk(slot, accs):
        def it_body(i, accs):
            base = pl.multiple_of(i, LANES * UNROLL)
            return tuple(
                accs[u] + (rows_v[slot, 0, pl.ds(base + u * LANES, LANES)]
                           * rows_v[slot, 1, pl.ds(base + u * LANES, LANES)])
                for u in range(UNROLL))
        return plsc.parallel_loop(
            0, CHUNK, LANES * UNROLL, unroll=2, carry=accs)(it_body)

    accs = tuple(jnp.zeros((LANES,), jnp.float32) for _ in range(UNROLL))
    dmas = [start(0, 0), None]
    for s in range(nsteps):
        slot = s % 2
        if s + 1 < nsteps:
            dmas[1 - slot] = start(s + 1, 1 - slot)
        dmas[slot].wait()
        accs = fma_chunk(slot, accs)

    acc = accs[0]
    for u in range(1, UNROLL):
        acc = acc + accs[u]
    acc_v[...] = acc * (PENALTY / batch)
    pltpu.sync_copy(acc_v, out_hbm.at[wid])


def kernel(compositions, pair_indices_i, pair_indices_j):
    batch = compositions.shape[0]
    comp_t = compositions.T

    info = plsc.get_sparse_core_info()
    nc, ns = info.num_cores, info.num_subcores
    nw = nc * ns
    pairs_per_w = PAIRS // nw

    mesh = plsc.VectorSubcoreMesh(core_axis_name="c", subcore_axis_name="s")
    run = pl.kernel(
        functools.partial(_body, batch, pairs_per_w, nc),
        out_type=jax.ShapeDtypeStruct((nw, LANES), jnp.float32),
        mesh=mesh,
        compiler_params=pltpu.CompilerParams(needs_layout_passes=False),
        scratch_types=[
            pltpu.VMEM((PAIRS,), jnp.int32),
            pltpu.VMEM((PAIRS,), jnp.int32),
            [pltpu.VMEM((LANES,), jnp.int32) for _ in range(pairs_per_w)],
            pltpu.VMEM((2, 2, CHUNK), jnp.float32),
            pltpu.VMEM((LANES,), jnp.float32),
            pltpu.SemaphoreType.DMA,
            pltpu.SemaphoreType.DMA,
        ],
    )
    partials = run(comp_t,
                   pair_indices_i.astype(jnp.int32),
                   pair_indices_j.astype(jnp.int32))
    return jnp.sum(partials)


# speedup vs baseline: 1.1291x; 1.1291x over previous
import functools

import jax
import jax.numpy as jnp
from jax import lax
from jax.experimental import pallas as pl
from jax.experimental.pallas import tpu as pltpu
from jax.experimental.pallas import tpu_sc as plsc

PAIRS = 128
LANES = 16
PENALTY = 10.0
CHUNK = 2048
NBUF = 4
UNROLL = 8


def _body(batch, pairs_per_w, nc,
          comp_hbm, idx_i_hbm, idx_j_hbm, out_hbm,
          ii_v, jj_v, iv_v, rows_v, acc_v, *sems):
    wid = lax.axis_index("s") * nc + lax.axis_index("c")
    p0 = wid * pairs_per_w
    nrows = 2 * pairs_per_w

    pltpu.sync_copy(idx_i_hbm, ii_v)
    pltpu.sync_copy(idx_j_hbm, jj_v)

    iota = lax.iota(jnp.int32, LANES)
    pv = p0 + iota // 2
    gi = plsc.load_gather(ii_v, [pv])
    gj = plsc.load_gather(jj_v, [pv])
    iv_v[...] = jnp.where(iota % 2 == 0, gi, gj)

    nsteps = batch // CHUNK

    def start(step, slot):
        return pltpu.async_copy(
            comp_hbm.at[iv_v.at[pl.ds(0, nrows)], pl.ds(step * CHUNK, CHUNK)],
            rows_v.at[slot], sems[slot])

    def fma_chunk(slot, accs):
        stride = LANES * UNROLL // pairs_per_w

        def it_body(it, accs):
            base = pl.multiple_of(it * stride, stride)
            new = list(accs)
            for u in range(UNROLL):
                k, c = divmod(u, UNROLL // pairs_per_w)
                off = base + c * LANES
                new[u] = accs[u] + (rows_v[slot, 2 * k, pl.ds(off, LANES)]
                                    * rows_v[slot, 2 * k + 1, pl.ds(off, LANES)])
            return tuple(new)
        return lax.fori_loop(0, CHUNK // stride, it_body, accs)

    accs = tuple(jnp.zeros((LANES,), jnp.float32) for _ in range(UNROLL))
    dmas = [None] * NBUF
    for b in range(min(NBUF, nsteps)):
        dmas[b] = start(b, b)
    for s in range(nsteps):
        slot = s % NBUF
        dmas[slot].wait()
        accs = fma_chunk(slot, accs)
        if s + NBUF < nsteps:
            dmas[slot] = start(s + NBUF, slot)

    acc = accs[0]
    for u in range(1, UNROLL):
        acc = acc + accs[u]
    acc_v[...] = acc * (PENALTY / batch)
    pltpu.sync_copy(acc_v, out_hbm.at[wid])


def kernel(compositions, pair_indices_i, pair_indices_j):
    batch = compositions.shape[0]
    comp_t = compositions.T

    info = plsc.get_sparse_core_info()
    nc, ns = info.num_cores, info.num_subcores
    nw = nc * ns
    pairs_per_w = PAIRS // nw

    mesh = plsc.VectorSubcoreMesh(core_axis_name="c", subcore_axis_name="s")
    run = pl.kernel(
        functools.partial(_body, batch, pairs_per_w, nc),
        out_type=jax.ShapeDtypeStruct((nw, LANES), jnp.float32),
        mesh=mesh,
        compiler_params=pltpu.CompilerParams(needs_layout_passes=False),
        scratch_types=[
            pltpu.VMEM((PAIRS,), jnp.int32),
            pltpu.VMEM((PAIRS,), jnp.int32),
            pltpu.VMEM((LANES,), jnp.int32),
            pltpu.VMEM((NBUF, 2 * pairs_per_w, CHUNK), jnp.float32),
            pltpu.VMEM((LANES,), jnp.float32),
        ] + [pltpu.SemaphoreType.DMA] * NBUF,
    )
    partials = run(comp_t,
                   pair_indices_i.astype(jnp.int32),
                   pair_indices_j.astype(jnp.int32))
    return jnp.sum(partials)
